# baseline (device time: 19447 ns/iter reference)
import jax
import jax.numpy as jnp
from jax import lax
from jax.experimental import pallas as pl
from jax.experimental.pallas import tpu as pltpu

B, H, D, BS = 16, 16, 64, 16
NSLOTS = 128
NP = 128
R = H * B
HD = H * D
G = 4
HG = H // G
CW = HG * D
RG = HG * B
NC = 4
TC = BS // NC


def kernel(Q, K, V, bt, lens):
    lens2 = lens.reshape(B, 1)
    q2 = Q.reshape(B, HD)
    k3 = K.transpose(1, 2, 3, 0).reshape(BS, HD, NP)
    v3 = V.transpose(1, 2, 3, 0).reshape(BS, HD, NP)

    def body(q_ref, k_ref, v_ref, bt_ref, lens_ref, out_ref,
             s_ref, ck_scr, qb_scr, mparts, lparts, oparts,
             msend, lsend, osend, mrecv, lrecv, orecv,
             sem_sm, sem_rm, sem_sl, sem_rl, sem_so, sem_ro):
        i = pl.program_id(0)
        my_x = lax.axis_index("x")
        my_y = lax.axis_index("y")
        nbr = (1 - my_x, my_y)
        barrier = pltpu.get_barrier_semaphore()

        @pl.when(i == 0)
        def _():
            pl.semaphore_signal(barrier, inc=1, device_id=nbr,
                                device_id_type=pl.DeviceIdType.MESH)
            slot = lax.broadcasted_iota(jnp.int32, (B, NSLOTS, NP), 1)
            page = lax.broadcasted_iota(jnp.int32, (B, NSLOTS, NP), 2)
            btl = bt_ref[...] - my_x * NP
            hit = ((btl[:, :, None] == page)
                   & (slot < lens_ref[...][:, :, None]))
            ck_scr[...] = jnp.sum(hit.astype(jnp.float32), axis=1)
            for g in range(G):
                q_g = q_ref[:, g * CW:(g + 1) * CW]
                qrep = jnp.concatenate([q_g] * HG, axis=0)
                rowh = lax.broadcasted_iota(jnp.int32, (RG, CW), 0) // B
                colh = lax.broadcasted_iota(jnp.int32, (RG, CW), 1) // D
                qb_scr[g, :, :] = jnp.where(
                    rowh == colh, qrep, 0.0).astype(jnp.bfloat16)

        kb = k_ref[...].astype(jnp.bfloat16)
        vb = v_ref[...].astype(jnp.bfloat16)

        for t in range(TC):
            for g in range(G):
                s_ref[t, g * RG:(g + 1) * RG, :] = lax.dot_general(
                    qb_scr[g], kb[t, g * CW:(g + 1) * CW, :],
                    (((1,), (0,)), ((), ())),
                    preferred_element_type=jnp.float32) * (D ** -0.5)

        s4 = s_ref[...].reshape(TC, H, B, NP)
        m_i = jnp.max(jnp.max(s4, axis=3), axis=0)
        p4 = (jnp.exp(s4 - m_i[None, :, :, None])
              * ck_scr[...][None, None, :, :])
        l_i = jnp.sum(jnp.sum(p4, axis=3), axis=0)
        mparts[i, :, :] = m_i
        lparts[i, :, :] = l_i

        pb = p4.reshape(TC, R, NP).astype(jnp.bfloat16)
        for g in range(G):
            o_g = lax.dot_general(
                pb[0, g * RG:(g + 1) * RG, :],
                vb[0, g * CW:(g + 1) * CW, :],
                (((1,), (1,)), ((), ())),
                preferred_element_type=jnp.float32)
            for t in range(1, TC):
                o_g = o_g + lax.dot_general(
                    pb[t, g * RG:(g + 1) * RG, :],
                    vb[t, g * CW:(g + 1) * CW, :],
                    (((1,), (1,)), ((), ())),
                    preferred_element_type=jnp.float32)
            for hl in range(HG):
                oparts[i, g * HG + hl, :, :] = (
                    o_g[hl * B:(hl + 1) * B, hl * D:(hl + 1) * D])

        @pl.when(i == NC - 1)
        def _():
            m_all = mparts[...]
            m_loc = jnp.max(m_all, axis=0)
            a = jnp.exp(m_all - m_loc[None, :, :])
            l_loc = jnp.sum(lparts[...] * a, axis=0)
            o_loc = jnp.sum(oparts[...] * a[:, :, :, None], axis=0)
            msend[...] = m_loc
            lsend[...] = l_loc
            osend[...] = o_loc

            pl.semaphore_wait(barrier, 1)
            rdma_m = pltpu.make_async_remote_copy(
                src_ref=msend, dst_ref=mrecv, send_sem=sem_sm,
                recv_sem=sem_rm, device_id=nbr,
                device_id_type=pl.DeviceIdType.MESH)
            rdma_l = pltpu.make_async_remote_copy(
                src_ref=lsend, dst_ref=lrecv, send_sem=sem_sl,
                recv_sem=sem_rl, device_id=nbr,
                device_id_type=pl.DeviceIdType.MESH)
            rdma_o = pltpu.make_async_remote_copy(
                src_ref=osend, dst_ref=orecv, send_sem=sem_so,
                recv_sem=sem_ro, device_id=nbr,
                device_id_type=pl.DeviceIdType.MESH)
            rdma_m.start()
            rdma_l.start()
            rdma_o.start()
            rdma_m.wait()
            rdma_l.wait()
            rdma_o.wait()

            m_rem, l_rem = mrecv[...], lrecv[...]
            mm = jnp.maximum(m_loc, m_rem)
            a_loc = jnp.exp(m_loc - mm)
            a_rem = jnp.exp(m_rem - mm)
            ll = l_loc * a_loc + l_rem * a_rem
            oo = (o_loc * a_loc[:, :, None]
                  + orecv[...] * a_rem[:, :, None]) / ll[:, :, None]
            out_ref[...] = jnp.swapaxes(oo, 0, 1).reshape(B, 1, H, D)

    return pl.pallas_call(
        body,
        grid=(NC,),
        out_shape=jax.ShapeDtypeStruct((B, 1, H, D), jnp.float32),
        in_specs=[
            pl.BlockSpec((B, HD), lambda i: (0, 0)),
            pl.BlockSpec((TC, HD, NP), lambda i: (i, 0, 0)),
            pl.BlockSpec((TC, HD, NP), lambda i: (i, 0, 0)),
            pl.BlockSpec((B, NSLOTS), lambda i: (0, 0)),
            pl.BlockSpec((B, 1), lambda i: (0, 0)),
        ],
        out_specs=pl.BlockSpec((B, 1, H, D), lambda i: (0, 0, 0, 0)),
        scratch_shapes=[
            pltpu.VMEM((TC, R, NP), jnp.float32),
            pltpu.VMEM((B, NP), jnp.float32),
            pltpu.VMEM((G, RG, CW), jnp.bfloat16),
            pltpu.VMEM((NC, H, B), jnp.float32),
            pltpu.VMEM((NC, H, B), jnp.float32),
            pltpu.VMEM((NC, H, B, D), jnp.float32),
            pltpu.VMEM((H, B), jnp.float32),
            pltpu.VMEM((H, B), jnp.float32),
            pltpu.VMEM((H, B, D), jnp.float32),
            pltpu.VMEM((H, B), jnp.float32),
            pltpu.VMEM((H, B), jnp.float32),
            pltpu.VMEM((H, B, D), jnp.float32),
            pltpu.SemaphoreType.DMA,
            pltpu.SemaphoreType.DMA,
            pltpu.SemaphoreType.DMA,
            pltpu.SemaphoreType.DMA,
            pltpu.SemaphoreType.DMA,
            pltpu.SemaphoreType.DMA,
        ],
        compiler_params=pltpu.CompilerParams(
            collective_id=0, vmem_limit_bytes=100 * 1024 * 1024),
    )(q2, k3, v3, bt, lens2)
